# baseline (speedup 1.0000x reference)
import functools

import jax
import jax.numpy as jnp
from jax.experimental import pallas as pl
from jax.experimental.pallas import tpu as pltpu

_LANE = 128
_VMEM_LIMIT_BYTES = 60 * 1024 * 1024
_NBUF = 3
_CHUNK_ROWS = 16


def _round_up(x, m):
    return ((x + m - 1) // m) * m


def _manual_kernel(x_any, inv_ref, w1_ref, b1_ref, w2_ref, b2_ref, out_ref,
                   buf, pooled, sems, *, rows_per_core, chunk_rows):
    pid = pl.program_id(0)
    base = pid * rows_per_core
    nch = rows_per_core // chunk_rows

    def start(k):
        slot = jax.lax.rem(k, _NBUF)
        pltpu.make_async_copy(
            x_any.at[pl.ds(base + k * chunk_rows, chunk_rows)],
            buf.at[slot],
            sems.at[slot],
        ).start()

    for s in range(min(_NBUF, nch)):
        start(s)

    def loop_body(k, _):
        slot = jax.lax.rem(k, _NBUF)
        pltpu.make_async_copy(
            x_any.at[pl.ds(base + k * chunk_rows, chunk_rows)],
            buf.at[slot],
            sems.at[slot],
        ).wait()
        pooled[pl.ds(k * chunk_rows, chunk_rows), :] = jnp.sum(buf[slot], axis=1)

        @pl.when(k + _NBUF < nch)
        def _():
            start(k + _NBUF)

        return 0

    jax.lax.fori_loop(0, nch, loop_body, 0)

    x = pooled[...] * inv_ref[...]
    h = jnp.dot(x, w1_ref[...], preferred_element_type=jnp.float32) + b1_ref[...]
    h = jnp.tanh(h)
    y = jnp.dot(h, w2_ref[...], preferred_element_type=jnp.float32) + b2_ref[...]
    out_ref[...] = y


def kernel(features, w1, b1, w2, b2, src_lengths):
    B, T, D = features.shape
    inner = w1.shape[1]
    C = w2.shape[1]

    b1 = jnp.reshape(b1, (1, inner)).astype(jnp.float32)
    b2 = jnp.reshape(b2, (1, C)).astype(jnp.float32)

    c_pad = _round_up(C, _LANE)
    if c_pad != C:
        w2 = jnp.pad(w2, ((0, 0), (0, c_pad - C)))
        b2 = jnp.pad(b2, ((0, 0), (0, c_pad - C)))

    ncores = 2
    rows = B // ncores
    inv_len = (1.0 / src_lengths.astype(jnp.float32)).reshape(B, 1)

    fn = functools.partial(_manual_kernel, rows_per_core=rows,
                           chunk_rows=_CHUNK_ROWS)
    out = pl.pallas_call(
        fn,
        out_shape=jax.ShapeDtypeStruct((B, c_pad), jnp.float32),
        grid_spec=pltpu.PrefetchScalarGridSpec(
            num_scalar_prefetch=0,
            grid=(ncores,),
            in_specs=[
                pl.BlockSpec(memory_space=pl.ANY),
                pl.BlockSpec((rows, 1), lambda i: (i, 0)),
                pl.BlockSpec((D, inner), lambda i: (0, 0)),
                pl.BlockSpec((1, inner), lambda i: (0, 0)),
                pl.BlockSpec((inner, c_pad), lambda i: (0, 0)),
                pl.BlockSpec((1, c_pad), lambda i: (0, 0)),
            ],
            out_specs=pl.BlockSpec((rows, c_pad), lambda i: (i, 0)),
            scratch_shapes=[
                pltpu.VMEM((_NBUF, _CHUNK_ROWS, T, D), jnp.float32),
                pltpu.VMEM((rows, D), jnp.float32),
                pltpu.SemaphoreType.DMA((_NBUF,)),
            ],
        ),
        compiler_params=pltpu.CompilerParams(
            dimension_semantics=("parallel",),
            vmem_limit_bytes=_VMEM_LIMIT_BYTES,
        ),
    )(features, inv_len, w1, b1, w2, b2)

    return out[:B, :C].astype(features.dtype)


# speedup vs baseline: 1.0627x; 1.0627x over previous
import jax
import jax.numpy as jnp
from jax.experimental import pallas as pl
from jax.experimental.pallas import tpu as pltpu

_LANE = 128
_VMEM_LIMIT_BYTES = 48 * 1024 * 1024


def _round_up(x, m):
    return ((x + m - 1) // m) * m


def _head_kernel(x_ref, inv_ref, w1_ref, b1_ref, w2_ref, b2_ref, out_ref):
    s = jnp.sum(x_ref[...], axis=1)
    x = s * inv_ref[...]
    h = jnp.dot(x, w1_ref[...], preferred_element_type=jnp.float32) + b1_ref[...]
    h = jnp.tanh(h)
    y = jnp.dot(h, w2_ref[...], preferred_element_type=jnp.float32) + b2_ref[...]
    out_ref[...] = y


def kernel(features, w1, b1, w2, b2, src_lengths):
    B, T, D = features.shape
    inner = w1.shape[1]
    C = w2.shape[1]

    b1 = jnp.reshape(b1, (1, inner)).astype(jnp.float32)
    b2 = jnp.reshape(b2, (1, C)).astype(jnp.float32)

    c_pad = _round_up(C, _LANE)
    if c_pad != C:
        w2 = jnp.pad(w2, ((0, 0), (0, c_pad - C)))
        b2 = jnp.pad(b2, ((0, 0), (0, c_pad - C)))

    tb = 16
    b_pad = _round_up(B, tb)
    if b_pad != B:
        features = jnp.pad(features, ((0, b_pad - B), (0, 0), (0, 0)))
    nb = b_pad // tb

    inv_len = (1.0 / src_lengths.astype(jnp.float32)).reshape(B, 1)
    if b_pad != B:
        inv_len = jnp.pad(inv_len, ((0, b_pad - B), (0, 0)), constant_values=1.0)

    out = pl.pallas_call(
        _head_kernel,
        out_shape=jax.ShapeDtypeStruct((b_pad, c_pad), jnp.float32),
        grid_spec=pltpu.PrefetchScalarGridSpec(
            num_scalar_prefetch=0,
            grid=(nb,),
            in_specs=[
                pl.BlockSpec((tb, T, D), lambda i: (i, 0, 0)),
                pl.BlockSpec((tb, 1), lambda i: (i, 0)),
                pl.BlockSpec((D, inner), lambda i: (0, 0)),
                pl.BlockSpec((1, inner), lambda i: (0, 0)),
                pl.BlockSpec((inner, c_pad), lambda i: (0, 0)),
                pl.BlockSpec((1, c_pad), lambda i: (0, 0)),
            ],
            out_specs=pl.BlockSpec((tb, c_pad), lambda i: (i, 0)),
        ),
        compiler_params=pltpu.CompilerParams(
            dimension_semantics=("parallel",),
            vmem_limit_bytes=_VMEM_LIMIT_BYTES,
        ),
    )(features, inv_len, w1, b1, w2, b2)

    return out[:B, :C].astype(features.dtype)
